# baseline (speedup 1.0000x reference)
import functools

import jax
import jax.numpy as jnp
from jax import lax
from jax.experimental import pallas as pl
from jax.experimental.pallas import tpu as pltpu
from jax.experimental.pallas import tpu_sc as plsc

HID = 64
SCALE = 8.0

_info = plsc.get_sparse_core_info()
NC, NS, L = _info.num_cores, _info.num_subcores, _info.num_lanes
NW = NC * NS

CH = 128
NBUF = 4
NOB = 2

FW = 1024


@jax.jit
def _fuse_table(emb):
    V = emb.shape[0]
    grid = (V + FW - 1) // FW
    nrow = grid * (FW // 2)

    def body(a_ref, o_ref):
        a = a_ref[...] * SCALE
        for sb in range(FW // 256):
            r0 = sb * 128
            c0 = sb * 256
            o_ref[r0 : r0 + 128, 0:HID] = a[c0 : c0 + 128, :]
            o_ref[r0 : r0 + 128, HID : 2 * HID] = a[c0 + 128 : c0 + 256, :]

    return pl.pallas_call(
        body,
        grid=(grid,),
        in_specs=[pl.BlockSpec((FW, HID), lambda i: (i, 0))],
        out_specs=pl.BlockSpec((FW // 2, 2 * HID), lambda i: (i, 0)),
        out_shape=jax.ShapeDtypeStruct((nrow, 2 * HID), jnp.float32),
    )(emb)


@functools.partial(jax.jit, static_argnames=("nchunk",))
def _emb_lookup(x3d, table2, *, nchunk):
    B = NW * nchunk * CH
    mesh = plsc.VectorSubcoreMesh(core_axis_name="c", subcore_axis_name="s")

    @functools.partial(
        pl.kernel,
        mesh=mesh,
        out_type=jax.ShapeDtypeStruct((B, HID), jnp.float32),
        scratch_types=[
            pltpu.VMEM((nchunk, CH), jnp.int32),
            [pltpu.VMEM((CH,), jnp.int32)] * NBUF,
            [pltpu.VMEM((CH, 2 * HID), jnp.float32)] * NBUF,
            [pltpu.VMEM((CH, HID), jnp.float32)] * NOB,
            [pltpu.SemaphoreType.DMA] * NBUF,
            [pltpu.SemaphoreType.DMA] * NOB,
        ],
    )
    def body(x_hbm, tab_hbm, out_hbm, idx_v, idx2, rows, cbuf, gsem, osem):
        wid = lax.axis_index("s") * NC + lax.axis_index("c")
        base = wid * (nchunk * CH)
        pltpu.sync_copy(x_hbm.at[wid], idx_v)

        def issue_gather(c, b):
            for j in range(CH // L):
                v = idx_v[c, pl.ds(j * L, L)]
                idx2[b][pl.ds(j * L, L)] = ((v >> 8) << 7) | (v & 127)
            pltpu.async_copy(tab_hbm.at[idx2[b]], rows[b], gsem[b])

        def wait_gather(b):
            pltpu.make_async_copy(
                tab_hbm.at[idx2[b]], rows[b], gsem[b]
            ).wait()

        def out_slice(c):
            return out_hbm.at[pl.ds(base + c * CH, CH)]

        def issue_scatter(c, ob):
            pltpu.async_copy(cbuf[ob], out_slice(c), osem[ob])

        def wait_scatter(c, ob):
            pltpu.make_async_copy(cbuf[ob], out_slice(c), osem[ob]).wait()

        def select_half(c, b, ob):
            buf = rows[b]
            cb = cbuf[ob]

            @plsc.parallel_loop(0, CH // L, unroll=1)
            def _grp(t):
                offv = (idx_v[c, pl.ds(t * L, L)] >> 7) & 1
                for i in range(L):
                    r = t * L + i
                    odd = offv[i] == 1
                    for s in range(HID // L):
                        lo = buf[r, pl.ds(s * L, L)]
                        hi = buf[r, pl.ds(HID + s * L, L)]
                        cb[r, pl.ds(s * L, L)] = jnp.where(odd, hi, lo)

        for c in range(NBUF - 1):
            issue_gather(c, c)

        ngroup = nchunk // NBUF

        def group(g, _):
            c0 = g * NBUF
            for b in range(NBUF):
                c = c0 + b
                wait_gather(b)
                ob = b % NOB

                @pl.when(c >= NOB)
                def _ws():
                    wait_scatter(c - NOB, ob)

                select_half(c, b, ob)
                issue_scatter(c, ob)

                @pl.when(c + NBUF - 1 < nchunk)
                def _ig():
                    issue_gather(c + NBUF - 1, (b + NBUF - 1) % NBUF)

            return 0

        lax.fori_loop(0, ngroup, group, 0)

        for b in range(NBUF - NOB, NBUF):
            wait_scatter(nchunk - NBUF + b, b % NOB)

    return body(x3d, table2)


def kernel(x, emb_weight):
    b, s = x.shape
    total = b * s
    nchunk = total // (NW * CH)
    x3d = x.reshape(-1).astype(jnp.int32).reshape(NW, nchunk, CH)
    table2 = _fuse_table(emb_weight)
    out = _emb_lookup(x3d, table2, nchunk=nchunk)
    return out.reshape(b, s, HID)


# speedup vs baseline: 1.2691x; 1.2691x over previous
import functools

import jax
import jax.numpy as jnp
from jax import lax
from jax.experimental import pallas as pl
from jax.experimental.pallas import tpu as pltpu
from jax.experimental.pallas import tpu_sc as plsc

HID = 64
SCALE = 8.0

_info = plsc.get_sparse_core_info()
NC, NS, L = _info.num_cores, _info.num_subcores, _info.num_lanes
NW = NC * NS

CH = 128
NBUF = 4
NOB = 2

FW = 1024


@jax.jit
def _fuse_table(embT):
    V = embT.shape[1]
    grid = (V + FW - 1) // FW
    nrow = grid * (FW // 2)

    def body(a_ref, o_ref):
        a = a_ref[...] * SCALE
        at = a.T
        for sb in range(FW // 256):
            r0 = sb * 128
            c0 = sb * 256
            o_ref[r0 : r0 + 128, 0:HID] = at[c0 : c0 + 128, :]
            o_ref[r0 : r0 + 128, HID : 2 * HID] = at[c0 + 128 : c0 + 256, :]

    return pl.pallas_call(
        body,
        grid=(grid,),
        in_specs=[pl.BlockSpec((HID, FW), lambda i: (0, i))],
        out_specs=pl.BlockSpec((FW // 2, 2 * HID), lambda i: (i, 0)),
        out_shape=jax.ShapeDtypeStruct((nrow, 2 * HID), jnp.float32),
    )(embT)


@functools.partial(jax.jit, static_argnames=("nchunk",))
def _emb_lookup(x3d, table2, *, nchunk):
    B = NW * nchunk * CH
    mesh = plsc.VectorSubcoreMesh(core_axis_name="c", subcore_axis_name="s")

    @functools.partial(
        pl.kernel,
        mesh=mesh,
        out_type=jax.ShapeDtypeStruct((B, HID), jnp.float32),
        scratch_types=[
            pltpu.VMEM((nchunk, CH), jnp.int32),
            [pltpu.VMEM((CH,), jnp.int32)] * NBUF,
            [pltpu.VMEM((CH, 2 * HID), jnp.float32)] * NBUF,
            [pltpu.VMEM((CH, HID), jnp.float32)] * NOB,
            [pltpu.SemaphoreType.DMA] * NBUF,
            [pltpu.SemaphoreType.DMA] * NOB,
        ],
    )
    def body(x_hbm, tab_hbm, out_hbm, idx_v, idx2, rows, cbuf, gsem, osem):
        wid = lax.axis_index("s") * NC + lax.axis_index("c")
        base = wid * (nchunk * CH)
        pltpu.sync_copy(x_hbm.at[wid], idx_v)

        def issue_gather(c, b):
            for j in range(CH // L):
                v = idx_v[c, pl.ds(j * L, L)]
                idx2[b][pl.ds(j * L, L)] = ((v >> 8) << 7) | (v & 127)
            pltpu.async_copy(tab_hbm.at[idx2[b]], rows[b], gsem[b])

        def wait_gather(b):
            pltpu.make_async_copy(
                tab_hbm.at[idx2[b]], rows[b], gsem[b]
            ).wait()

        def out_slice(c):
            return out_hbm.at[pl.ds(base + c * CH, CH)]

        def issue_scatter(c, ob):
            pltpu.async_copy(cbuf[ob], out_slice(c), osem[ob])

        def wait_scatter(c, ob):
            pltpu.make_async_copy(cbuf[ob], out_slice(c), osem[ob]).wait()

        def select_half(c, b, ob):
            buf = rows[b]
            cb = cbuf[ob]

            @plsc.parallel_loop(0, CH // L, unroll=1)
            def _grp(t):
                offv = (idx_v[c, pl.ds(t * L, L)] >> 7) & 1
                for i in range(L):
                    r = t * L + i
                    odd = offv[i] == 1
                    for s in range(HID // L):
                        lo = buf[r, pl.ds(s * L, L)]
                        hi = buf[r, pl.ds(HID + s * L, L)]
                        cb[r, pl.ds(s * L, L)] = jnp.where(odd, hi, lo)

        for c in range(NBUF - 1):
            issue_gather(c, c)

        ngroup = nchunk // NBUF

        def group(g, _):
            c0 = g * NBUF
            for b in range(NBUF):
                c = c0 + b
                wait_gather(b)
                ob = b % NOB

                @pl.when(c >= NOB)
                def _ws():
                    wait_scatter(c - NOB, ob)

                select_half(c, b, ob)
                issue_scatter(c, ob)

                @pl.when(c + NBUF - 1 < nchunk)
                def _ig():
                    issue_gather(c + NBUF - 1, (b + NBUF - 1) % NBUF)

            return 0

        lax.fori_loop(0, ngroup, group, 0)

        for b in range(NBUF - NOB, NBUF):
            wait_scatter(nchunk - NBUF + b, b % NOB)

    return body(x3d, table2)


def kernel(x, emb_weight):
    b, s = x.shape
    total = b * s
    nchunk = total // (NW * CH)
    x3d = x.reshape(-1).astype(jnp.int32).reshape(NW, nchunk, CH)
    table2 = _fuse_table(emb_weight.T)
    out = _emb_lookup(x3d, table2, nchunk=nchunk)
    return out.reshape(b, s, HID)
